# baseline (speedup 1.0000x reference)
import functools

import jax
import jax.numpy as jnp
from jax.experimental import pallas as pl
from jax.experimental.pallas import tpu as pltpu

_B = 16
_T = 16
_N = 32768
_BN = 2048
_CB = _BN // 128
_NT = jnp.float32


def _dotT(a, b):
    return jax.lax.dot_general(a, b, (((1,), (1,)), ((), ())),
                               preferred_element_type=jnp.float32)


def _body(xn_ref, cn_ref, segn_ref, xs_ref, cs_ref, segs_ref, C_ref,
          Fnb_ref, Fsg_ref, P_ref, S_ref,
          cntn_ref, cnts_ref):
    i = pl.program_id(0)
    C = C_ref
    BD1pT = C[:, 0:512]
    BD1nT = C[:, 512:1024]
    BD1sT = C[:, 1024:1536]
    BD2T = C[:, 1536:2048]
    BDnT = C[:, 2048:2112]
    BDsT = C[:, 2176:2208]
    BDpT = C[:, 2304:2368]
    ErbnT = C[:, 2432:2448]
    ErbsT = C[:, 2560:2576]
    R4 = C[0:64, 2688:2704]
    R2 = C[0:32, 2816:2832]
    xpedT = C[0:64, 2944:2960]
    BD3T = C[0:16, 3200:3712]
    bn_c = C[:, 3072:3073]
    bs_c = C[:, 3073:3074]
    bp_c = C[:, 3074:3075]
    b1_c = C[:, 3075:3076]
    b2_c = C[:, 3076:3077]
    b3_c = C[0:16, 3077:3078]

    def agg(x_ref, c_ref, seg_ref, BDT, b_c, R):
        m = (c_ref[...] != -1.0).astype(_NT)
        m4 = jnp.dot(R, m, preferred_element_type=jnp.float32)
        v = x_ref[...]
        d = v.shape[2]
        x = jnp.transpose(v, (0, 2, 1, 3)).reshape(_T * d, _BN)
        xm = x * m4.astype(x.dtype)
        em = jnp.maximum(
            jnp.dot(BDT, xm, preferred_element_type=jnp.float32)
            + b_c, 0.0)
        seg = seg_ref[...]
        oh = (jax.lax.broadcasted_iota(jnp.int32, (_B, _BN), 0)
              == jnp.broadcast_to(seg, (_B, _BN))).astype(_NT)
        f = _dotT(em, oh)
        cnt = _dotT((1.0 - m), oh)
        return f, cnt

    fn, cn = agg(xn_ref, cn_ref, segn_ref, BDnT, bn_c, R4)
    fs, cs = agg(xs_ref, cs_ref, segs_ref, BDsT, bs_c, R2)

    @pl.when(i == 0)
    def _():
        Fnb_ref[...] = fn
        Fsg_ref[...] = fs
        cntn_ref[...] = cn
        cnts_ref[...] = cs

    @pl.when(i > 0)
    def _():
        Fnb_ref[...] += fn
        Fsg_ref[...] += fs
        cntn_ref[...] += cn
        cnts_ref[...] += cs

    @pl.when(i == pl.num_programs(0) - 1)
    def _():
        Fn = Fnb_ref[...] - jnp.dot(ErbnT, cntn_ref[...],
                                    preferred_element_type=jnp.float32)
        Fs = Fsg_ref[...] - jnp.dot(ErbsT, cnts_ref[...],
                                    preferred_element_type=jnp.float32)
        Fnb_ref[...] = Fn
        Fsg_ref[...] = Fs
        P = jnp.maximum(
            jnp.dot(BDpT, xpedT,
                    preferred_element_type=jnp.float32) + bp_c, 0.0)
        P_ref[...] = P
        A1 = jnp.maximum(
            jnp.dot(BD1pT, P, preferred_element_type=jnp.float32)
            + jnp.dot(BD1nT, Fn, preferred_element_type=jnp.float32)
            + jnp.dot(BD1sT, Fs, preferred_element_type=jnp.float32)
            + b1_c, 0.0)
        A2 = jnp.maximum(
            jnp.dot(BD2T, A1, preferred_element_type=jnp.float32)
            + b2_c, 0.0)
        S_ref[...] = (jnp.dot(BD3T, A2,
                              preferred_element_type=jnp.float32)
                      + b3_c)


@functools.partial(jax.jit, static_argnames=("interpret",))
def _run(x_ped, x_neighbor, x_sign, cls_neighbor, cls_sign,
         seg_neighbor, seg_sign,
         W_ped, b_ped, W_nb, b_nb, W_sg, b_sg,
         W1, b1, W2, b2, W3, b3, interpret=False):
    NB = _N // _BN
    eyeT = jnp.eye(_T, dtype=jnp.float32)

    def bdT(W):
        return jnp.kron(eyeT, W.T)

    NC = _N // 128
    xn4 = (x_neighbor.transpose(1, 0, 2).reshape(_T, NC, 128, 4)
           .transpose(0, 1, 3, 2))
    xs4 = (x_sign.transpose(1, 0, 2).reshape(_T, NC, 128, 2)
           .transpose(0, 1, 3, 2))
    xpedT = x_ped.transpose(1, 2, 0).reshape(_T * 4, _B)
    cnT = cls_neighbor.T
    csT = cls_sign.T
    segn2 = seg_neighbor.reshape(1, _N)
    segs2 = seg_sign.reshape(1, _N)

    def padc(a, w=128):
        return jnp.pad(a, ((0, 512 - a.shape[0]), (0, w - a.shape[1])))

    col = lambda b: jnp.tile(b, _T).reshape(-1, 1)
    bcols = jnp.concatenate(
        [col(b_nb), col(b_sg), col(b_ped), col(b1), col(b2),
         jnp.broadcast_to(b3, (512,)).reshape(512, 1)], axis=1)
    CONSTS = jnp.concatenate([
        bdT(W1[0:32]), bdT(W1[32:64]), bdT(W1[64:96]), bdT(W2),
        padc(bdT(W_nb)), padc(bdT(W_sg)), padc(bdT(W_ped)),
        padc(jnp.kron(eyeT, jax.nn.relu(b_nb)[:, None])),
        padc(jnp.kron(eyeT, jax.nn.relu(b_sg)[:, None])),
        padc(jnp.kron(eyeT, jnp.ones((4, 1), jnp.float32))),
        padc(jnp.kron(eyeT, jnp.ones((2, 1), jnp.float32))),
        padc(x_ped.transpose(1, 2, 0).reshape(_T * 4, _B)),
        padc(bcols),
        padc(bdT(W3), 512),
    ], axis=1)

    const = lambda shape: pl.BlockSpec(shape, lambda i: (0,) * len(shape))
    Fnb, Fsg, P, S = pl.pallas_call(
        _body,
        grid=(NB,),
        in_specs=[
            pl.BlockSpec((_T, _CB, 4, 128), lambda i: (0, i, 0, 0)),
            pl.BlockSpec((_T, _BN), lambda i: (0, i)),
            pl.BlockSpec((1, _BN), lambda i: (0, i)),
            pl.BlockSpec((_T, _CB, 2, 128), lambda i: (0, i, 0, 0)),
            pl.BlockSpec((_T, _BN), lambda i: (0, i)),
            pl.BlockSpec((1, _BN), lambda i: (0, i)),
            const((512, 3712)),
        ],
        out_specs=[
            const((512, _B)), const((512, _B)), const((512, _B)),
            const((_T, _B)),
        ],
        out_shape=[
            jax.ShapeDtypeStruct((512, _B), jnp.float32),
            jax.ShapeDtypeStruct((512, _B), jnp.float32),
            jax.ShapeDtypeStruct((512, _B), jnp.float32),
            jax.ShapeDtypeStruct((_T, _B), jnp.float32),
        ],
        scratch_shapes=[
            pltpu.VMEM((_T, _B), jnp.float32),
            pltpu.VMEM((_T, _B), jnp.float32),
        ],
        interpret=interpret,
    )(xn4, cnT, segn2, xs4, csT, segs2, CONSTS)

    int_det_score = S.T.reshape(_B, _T, 1)
    all_traffic = jnp.concatenate(
        [P.T.reshape(_B, _T, 32), Fnb.T.reshape(_B, _T, 32),
         Fsg.T.reshape(_B, _T, 32)], axis=-1)
    return (int_det_score, all_traffic)


def kernel(x_ped, x_neighbor, x_sign, cls_neighbor, cls_sign,
           seg_neighbor, seg_sign,
           W_ped, b_ped, W_nb, b_nb, W_sg, b_sg,
           W1, b1, W2, b2, W3, b3):
    return _run(x_ped, x_neighbor, x_sign, cls_neighbor, cls_sign,
                seg_neighbor, seg_sign,
                W_ped, b_ped, W_nb, b_nb, W_sg, b_sg,
                W1, b1, W2, b2, W3, b3)


# speedup vs baseline: 1.9178x; 1.9178x over previous
import functools

import jax
import jax.numpy as jnp
from jax.experimental import pallas as pl
from jax.experimental.pallas import tpu as pltpu

_B = 16
_T = 16
_N = 32768
_BN = 2048
_CB = _BN // 128
_NT = jnp.float32
_F32 = jnp.float32


def _dotT(a, b):
    return jax.lax.dot_general(a, b, (((1,), (1,)), ((), ())),
                               preferred_element_type=_F32)


def _iota2(shape, dim):
    return jax.lax.broadcasted_iota(jnp.int32, shape, dim)


def _dot(a, b):
    return jnp.dot(a, b, preferred_element_type=_F32)


def _s512(n):
    return ((_iota2((512, n), 0) % n) == _iota2((512, n), 1)).astype(_F32)


def _s512T(n):
    return (_iota2((n, 512), 0) == (_iota2((n, 512), 1) % n)).astype(_F32)


def _body(xn_ref, cn_ref, segn_ref, xs_ref, cs_ref, segs_ref,
          w1T_ref, w2_ref, w3T_ref, wn_ref, wp_ref, ws_ref, bstack_ref,
          xped_ref,
          Fnb_ref, Fsg_ref, P_ref, S_ref,
          cntn_ref, cnts_ref, BDnT_ref, BDsT_ref, bcol_ref):
    i = pl.program_id(0)

    @pl.when(i == 0)
    def _():
        S = _s512(32)
        bcol_ref[:, 0:6] = _dot(S, jnp.transpose(bstack_ref[...]))
        def bd_small(w, d):
            wtile = _dot(_dot(S, jnp.transpose(w)),
                         (_iota2((d, _T * d), 0)
                          == (_iota2((d, _T * d), 1) % d)).astype(_F32))
            km = ((_iota2((512, _T * d), 0) // 32)
                  == (_iota2((512, _T * d), 1) // d)).astype(_F32)
            return wtile * km
        BDnT_ref[...] = bd_small(wn_ref[...], 4)
        BDsT_ref[...] = bd_small(ws_ref[...], 2)

    def agg(x_ref, c_ref, seg_ref, BDT, b_c, rshift):
        m = (c_ref[...] != -1.0).astype(_NT)
        R = ((_iota2((BDT.shape[1], _T), 0) >> rshift)
             == _iota2((BDT.shape[1], _T), 1)).astype(_F32)
        m4 = _dot(R, m)
        v = x_ref[...]
        d = v.shape[2]
        x = jnp.transpose(v, (0, 2, 1, 3)).reshape(_T * d, _BN)
        xm = x * m4
        em = jnp.maximum(_dot(BDT, xm) + b_c, 0.0)
        seg = seg_ref[...]
        oh = (_iota2((_B, _BN), 0)
              == jnp.broadcast_to(seg, (_B, _BN))).astype(_NT)
        f = _dotT(em, oh)
        cnt = _dotT((1.0 - m), oh)
        return f, cnt

    fn, cn = agg(xn_ref, cn_ref, segn_ref, BDnT_ref[...],
                 bcol_ref[:, 0:1], 2)
    fs, cs = agg(xs_ref, cs_ref, segs_ref, BDsT_ref[...],
                 bcol_ref[:, 1:2], 1)

    @pl.when(i == 0)
    def _():
        Fnb_ref[...] = fn
        Fsg_ref[...] = fs
        cntn_ref[...] = cn
        cnts_ref[...] = cs

    @pl.when(i > 0)
    def _():
        Fnb_ref[...] += fn
        Fsg_ref[...] += fs
        cntn_ref[...] += cn
        cnts_ref[...] += cs

    @pl.when(i == pl.num_programs(0) - 1)
    def _():
        S = _s512(32)
        ST = _s512T(32)
        km512 = ((_iota2((512, 512), 0) // 32)
                 == (_iota2((512, 512), 1) // 32)).astype(_F32)

        def bd(wT):
            return _dot(_dot(S, wT), ST) * km512

        w1T = w1T_ref[...]
        maskE = ((_iota2((512, _B), 0) // 32)
                 == _iota2((512, _B), 1)).astype(_F32)
        ErbnT = jnp.maximum(bcol_ref[:, 0:1], 0.0) * maskE
        ErbsT = jnp.maximum(bcol_ref[:, 1:2], 0.0) * maskE
        Fn = Fnb_ref[...] - _dot(ErbnT, cntn_ref[...])
        Fs = Fsg_ref[...] - _dot(ErbsT, cnts_ref[...])
        Fnb_ref[...] = Fn
        Fsg_ref[...] = Fs

        wtile_p = _dot(_dot(S, jnp.transpose(wp_ref[...])),
                       (_iota2((4, 64), 0)
                        == (_iota2((4, 64), 1) % 4)).astype(_F32))
        km64 = ((_iota2((512, 64), 0) // 32)
                == (_iota2((512, 64), 1) // 4)).astype(_F32)
        BDpT = wtile_p * km64
        P = jnp.maximum(_dot(BDpT, xped_ref[...]) + bcol_ref[:, 2:3], 0.0)
        P_ref[...] = P

        A1 = jnp.maximum(
            bd(w1T[:, 0:32]) @ P + bd(w1T[:, 32:64]) @ Fn
            + bd(w1T[:, 64:96]) @ Fs + bcol_ref[:, 3:4], 0.0)
        A2 = jnp.maximum(bd(jnp.transpose(w2_ref[...])) @ A1
                         + bcol_ref[:, 4:5], 0.0)
        w3L = _dot(w3T_ref[...], ST)
        BD3T = (jnp.broadcast_to(w3L, (_B, 512))
                * (_iota2((_B, 512), 0)
                   == (_iota2((_B, 512), 1) // 32)).astype(_F32))
        b3v = jnp.broadcast_to(bstack_ref[5:6, 0:1], (_T, _B))
        S_ref[...] = _dot(BD3T, A2) + b3v


@functools.partial(jax.jit, static_argnames=("interpret",))
def _run(x_ped, x_neighbor, x_sign, cls_neighbor, cls_sign,
         seg_neighbor, seg_sign,
         W_ped, b_ped, W_nb, b_nb, W_sg, b_sg,
         W1, b1, W2, b2, W3, b3, interpret=False):
    NB = _N // _BN

    NC = _N // 128
    xn4 = (x_neighbor.transpose(1, 0, 2).reshape(_T, NC, 128, 4)
           .transpose(0, 1, 3, 2))
    xs4 = (x_sign.transpose(1, 0, 2).reshape(_T, NC, 128, 2)
           .transpose(0, 1, 3, 2))
    cnT = cls_neighbor.T
    csT = cls_sign.T
    segn2 = seg_neighbor.reshape(1, _N)
    segs2 = seg_sign.reshape(1, _N)
    xpedT = x_ped.transpose(1, 2, 0).reshape(_T * 4, _B)
    bstack = jnp.stack([b_nb, b_sg, b_ped, b1, b2,
                        jnp.pad(b3, (0, 31))])

    const = lambda shape: pl.BlockSpec(shape, lambda i: (0,) * len(shape))
    Fnb, Fsg, P, S = pl.pallas_call(
        _body,
        grid=(NB,),
        in_specs=[
            pl.BlockSpec((_T, _CB, 4, 128), lambda i: (0, i, 0, 0)),
            pl.BlockSpec((_T, _BN), lambda i: (0, i)),
            pl.BlockSpec((1, _BN), lambda i: (0, i)),
            pl.BlockSpec((_T, _CB, 2, 128), lambda i: (0, i, 0, 0)),
            pl.BlockSpec((_T, _BN), lambda i: (0, i)),
            pl.BlockSpec((1, _BN), lambda i: (0, i)),
            const((32, 96)),
            const((32, 32)),
            const((1, 32)),
            const((4, 32)),
            const((4, 32)),
            const((2, 32)),
            const((6, 32)),
            const((_T * 4, _B)),
        ],
        out_specs=[
            const((512, _B)), const((512, _B)), const((512, _B)),
            const((_T, _B)),
        ],
        out_shape=[
            jax.ShapeDtypeStruct((512, _B), jnp.float32),
            jax.ShapeDtypeStruct((512, _B), jnp.float32),
            jax.ShapeDtypeStruct((512, _B), jnp.float32),
            jax.ShapeDtypeStruct((_T, _B), jnp.float32),
        ],
        scratch_shapes=[
            pltpu.VMEM((_T, _B), jnp.float32),
            pltpu.VMEM((_T, _B), jnp.float32),
            pltpu.VMEM((512, 64), jnp.float32),
            pltpu.VMEM((512, 32), jnp.float32),
            pltpu.VMEM((512, 8), jnp.float32),
        ],
        interpret=interpret,
    )(xn4, cnT, segn2, xs4, csT, segs2,
      W1.T, W2, W3.T, W_nb, W_ped, W_sg, bstack, xpedT)

    int_det_score = S.T.reshape(_B, _T, 1)
    all_traffic = jnp.concatenate(
        [P.T.reshape(_B, _T, 32), Fnb.T.reshape(_B, _T, 32),
         Fsg.T.reshape(_B, _T, 32)], axis=-1)
    return (int_det_score, all_traffic)


def kernel(x_ped, x_neighbor, x_sign, cls_neighbor, cls_sign,
           seg_neighbor, seg_sign,
           W_ped, b_ped, W_nb, b_nb, W_sg, b_sg,
           W1, b1, W2, b2, W3, b3):
    return _run(x_ped, x_neighbor, x_sign, cls_neighbor, cls_sign,
                seg_neighbor, seg_sign,
                W_ped, b_ped, W_nb, b_nb, W_sg, b_sg,
                W1, b1, W2, b2, W3, b3)


# speedup vs baseline: 2.0498x; 1.0688x over previous
import functools

import jax
import jax.numpy as jnp
from jax.experimental import pallas as pl
from jax.experimental.pallas import tpu as pltpu

_B = 16
_T = 16
_N = 32768
_BN = 4096
_CB = _BN // 128
_NT = jnp.float32
_F32 = jnp.float32


def _dotT(a, b):
    return jax.lax.dot_general(a, b, (((1,), (1,)), ((), ())),
                               preferred_element_type=_F32)


def _iota2(shape, dim):
    return jax.lax.broadcasted_iota(jnp.int32, shape, dim)


def _dot(a, b):
    return jnp.dot(a, b, preferred_element_type=_F32)


def _s512(n):
    return ((_iota2((512, n), 0) % n) == _iota2((512, n), 1)).astype(_F32)


def _s512T(n):
    return (_iota2((n, 512), 0) == (_iota2((n, 512), 1) % n)).astype(_F32)


def _body(xn_ref, cn_ref, segn_ref, xs_ref, cs_ref, segs_ref,
          w1T_ref, w2_ref, w3T_ref, wn_ref, wp_ref, ws_ref, bstack_ref,
          xped_ref,
          Fnb_ref, Fsg_ref, P_ref, S_ref,
          cntn_ref, cnts_ref, BDnT_ref, BDsT_ref, bcol_ref):
    i = pl.program_id(0)

    @pl.when(i == 0)
    def _():
        S = _s512(32)
        bcol_ref[:, 0:6] = _dot(S, jnp.transpose(bstack_ref[...]))
        def bd_small(w, d):
            wtile = _dot(_dot(S, jnp.transpose(w)),
                         (_iota2((d, _T * d), 0)
                          == (_iota2((d, _T * d), 1) % d)).astype(_F32))
            km = ((_iota2((512, _T * d), 0) // 32)
                  == (_iota2((512, _T * d), 1) // d)).astype(_F32)
            return wtile * km
        BDnT_ref[...] = bd_small(wn_ref[...], 4)
        BDsT_ref[...] = bd_small(ws_ref[...], 2)

    def agg(x_ref, c_ref, seg_ref, BDT, b_c, rshift):
        m = (c_ref[...] != -1.0).astype(_NT)
        R = ((_iota2((BDT.shape[1], _T), 0) >> rshift)
             == _iota2((BDT.shape[1], _T), 1)).astype(_F32)
        m4 = _dot(R, m)
        v = x_ref[...]
        d = v.shape[2]
        x = jnp.transpose(v, (0, 2, 1, 3)).reshape(_T * d, _BN)
        xm = x * m4
        em = jnp.maximum(_dot(BDT, xm) + b_c, 0.0)
        seg = seg_ref[...]
        oh = (_iota2((_B, _BN), 0)
              == jnp.broadcast_to(seg, (_B, _BN))).astype(_NT)
        f = _dotT(em, oh)
        cnt = _dotT((1.0 - m), oh)
        return f, cnt

    fn, cn = agg(xn_ref, cn_ref, segn_ref, BDnT_ref[...],
                 bcol_ref[:, 0:1], 2)
    fs, cs = agg(xs_ref, cs_ref, segs_ref, BDsT_ref[...],
                 bcol_ref[:, 1:2], 1)

    @pl.when(i == 0)
    def _():
        Fnb_ref[...] = fn
        Fsg_ref[...] = fs
        cntn_ref[...] = cn
        cnts_ref[...] = cs

    @pl.when(i > 0)
    def _():
        Fnb_ref[...] += fn
        Fsg_ref[...] += fs
        cntn_ref[...] += cn
        cnts_ref[...] += cs

    @pl.when(i == pl.num_programs(0) - 1)
    def _():
        S = _s512(32)
        ST = _s512T(32)
        km512 = ((_iota2((512, 512), 0) // 32)
                 == (_iota2((512, 512), 1) // 32)).astype(_F32)

        def bd(wT):
            return _dot(_dot(S, wT), ST) * km512

        w1T = w1T_ref[...]
        maskE = ((_iota2((512, _B), 0) // 32)
                 == _iota2((512, _B), 1)).astype(_F32)
        ErbnT = jnp.maximum(bcol_ref[:, 0:1], 0.0) * maskE
        ErbsT = jnp.maximum(bcol_ref[:, 1:2], 0.0) * maskE
        Fn = Fnb_ref[...] - _dot(ErbnT, cntn_ref[...])
        Fs = Fsg_ref[...] - _dot(ErbsT, cnts_ref[...])
        Fnb_ref[...] = Fn
        Fsg_ref[...] = Fs

        wtile_p = _dot(_dot(S, jnp.transpose(wp_ref[...])),
                       (_iota2((4, 64), 0)
                        == (_iota2((4, 64), 1) % 4)).astype(_F32))
        km64 = ((_iota2((512, 64), 0) // 32)
                == (_iota2((512, 64), 1) // 4)).astype(_F32)
        BDpT = wtile_p * km64
        P = jnp.maximum(_dot(BDpT, xped_ref[...]) + bcol_ref[:, 2:3], 0.0)
        P_ref[...] = P

        A1 = jnp.maximum(
            bd(w1T[:, 0:32]) @ P + bd(w1T[:, 32:64]) @ Fn
            + bd(w1T[:, 64:96]) @ Fs + bcol_ref[:, 3:4], 0.0)
        A2 = jnp.maximum(bd(jnp.transpose(w2_ref[...])) @ A1
                         + bcol_ref[:, 4:5], 0.0)
        w3L = _dot(w3T_ref[...], ST)
        BD3T = (jnp.broadcast_to(w3L, (_B, 512))
                * (_iota2((_B, 512), 0)
                   == (_iota2((_B, 512), 1) // 32)).astype(_F32))
        b3v = jnp.broadcast_to(bstack_ref[5:6, 0:1], (_T, _B))
        S_ref[...] = _dot(BD3T, A2) + b3v


@functools.partial(jax.jit, static_argnames=("interpret",))
def _run(x_ped, x_neighbor, x_sign, cls_neighbor, cls_sign,
         seg_neighbor, seg_sign,
         W_ped, b_ped, W_nb, b_nb, W_sg, b_sg,
         W1, b1, W2, b2, W3, b3, interpret=False):
    NB = _N // _BN

    NC = _N // 128
    xn4 = (x_neighbor.transpose(1, 0, 2).reshape(_T, NC, 128, 4)
           .transpose(0, 1, 3, 2))
    xs4 = (x_sign.transpose(1, 0, 2).reshape(_T, NC, 128, 2)
           .transpose(0, 1, 3, 2))
    cnT = cls_neighbor.T
    csT = cls_sign.T
    segn2 = seg_neighbor.reshape(1, _N)
    segs2 = seg_sign.reshape(1, _N)
    xpedT = x_ped.transpose(1, 2, 0).reshape(_T * 4, _B)
    bstack = jnp.stack([b_nb, b_sg, b_ped, b1, b2,
                        jnp.pad(b3, (0, 31))])

    const = lambda shape: pl.BlockSpec(shape, lambda i: (0,) * len(shape))
    Fnb, Fsg, P, S = pl.pallas_call(
        _body,
        grid=(NB,),
        in_specs=[
            pl.BlockSpec((_T, _CB, 4, 128), lambda i: (0, i, 0, 0)),
            pl.BlockSpec((_T, _BN), lambda i: (0, i)),
            pl.BlockSpec((1, _BN), lambda i: (0, i)),
            pl.BlockSpec((_T, _CB, 2, 128), lambda i: (0, i, 0, 0)),
            pl.BlockSpec((_T, _BN), lambda i: (0, i)),
            pl.BlockSpec((1, _BN), lambda i: (0, i)),
            const((32, 96)),
            const((32, 32)),
            const((1, 32)),
            const((4, 32)),
            const((4, 32)),
            const((2, 32)),
            const((6, 32)),
            const((_T * 4, _B)),
        ],
        out_specs=[
            const((512, _B)), const((512, _B)), const((512, _B)),
            const((_T, _B)),
        ],
        out_shape=[
            jax.ShapeDtypeStruct((512, _B), jnp.float32),
            jax.ShapeDtypeStruct((512, _B), jnp.float32),
            jax.ShapeDtypeStruct((512, _B), jnp.float32),
            jax.ShapeDtypeStruct((_T, _B), jnp.float32),
        ],
        scratch_shapes=[
            pltpu.VMEM((_T, _B), jnp.float32),
            pltpu.VMEM((_T, _B), jnp.float32),
            pltpu.VMEM((512, 64), jnp.float32),
            pltpu.VMEM((512, 32), jnp.float32),
            pltpu.VMEM((512, 8), jnp.float32),
        ],
        interpret=interpret,
    )(xn4, cnT, segn2, xs4, csT, segs2,
      W1.T, W2, W3.T, W_nb, W_ped, W_sg, bstack, xpedT)

    int_det_score = S.T.reshape(_B, _T, 1)
    all_traffic = jnp.concatenate(
        [P.T.reshape(_B, _T, 32), Fnb.T.reshape(_B, _T, 32),
         Fsg.T.reshape(_B, _T, 32)], axis=-1)
    return (int_det_score, all_traffic)


def kernel(x_ped, x_neighbor, x_sign, cls_neighbor, cls_sign,
           seg_neighbor, seg_sign,
           W_ped, b_ped, W_nb, b_nb, W_sg, b_sg,
           W1, b1, W2, b2, W3, b3):
    return _run(x_ped, x_neighbor, x_sign, cls_neighbor, cls_sign,
                seg_neighbor, seg_sign,
                W_ped, b_ped, W_nb, b_nb, W_sg, b_sg,
                W1, b1, W2, b2, W3, b3)


# speedup vs baseline: 2.0859x; 1.0176x over previous
import functools

import jax
import jax.numpy as jnp
from jax.experimental import pallas as pl
from jax.experimental.pallas import tpu as pltpu

_B = 16
_T = 16
_N = 32768
_BN = 8192
_CB = _BN // 128
_NT = jnp.float32
_F32 = jnp.float32


def _dotT(a, b):
    return jax.lax.dot_general(a, b, (((1,), (1,)), ((), ())),
                               preferred_element_type=_F32)


def _iota2(shape, dim):
    return jax.lax.broadcasted_iota(jnp.int32, shape, dim)


def _dot(a, b):
    return jnp.dot(a, b, preferred_element_type=_F32)


def _s512(n):
    return ((_iota2((512, n), 0) % n) == _iota2((512, n), 1)).astype(_F32)


def _s512T(n):
    return (_iota2((n, 512), 0) == (_iota2((n, 512), 1) % n)).astype(_F32)


def _body(xn_ref, cn_ref, segn_ref, xs_ref, cs_ref, segs_ref,
          w1T_ref, w2_ref, w3T_ref, wn_ref, wp_ref, ws_ref, bstack_ref,
          xped_ref,
          Fnb_ref, Fsg_ref, P_ref, S_ref,
          cntn_ref, cnts_ref, BDnT_ref, BDsT_ref, bcol_ref):
    i = pl.program_id(0)

    @pl.when(i == 0)
    def _():
        S = _s512(32)
        bcol_ref[:, 0:6] = _dot(S, jnp.transpose(bstack_ref[...]))
        def bd_small(w, d):
            wtile = _dot(_dot(S, jnp.transpose(w)),
                         (_iota2((d, _T * d), 0)
                          == (_iota2((d, _T * d), 1) % d)).astype(_F32))
            km = ((_iota2((512, _T * d), 0) // 32)
                  == (_iota2((512, _T * d), 1) // d)).astype(_F32)
            return wtile * km
        BDnT_ref[...] = bd_small(wn_ref[...], 4)
        BDsT_ref[...] = bd_small(ws_ref[...], 2)

    def agg(x_ref, c_ref, seg_ref, BDT, b_c, rshift):
        m = (c_ref[...] != -1.0).astype(_NT)
        R = ((_iota2((BDT.shape[1], _T), 0) >> rshift)
             == _iota2((BDT.shape[1], _T), 1)).astype(_F32)
        m4 = _dot(R, m)
        v = x_ref[...]
        d = v.shape[2]
        x = jnp.transpose(v, (0, 2, 1, 3)).reshape(_T * d, _BN)
        xm = x * m4
        em = jnp.maximum(_dot(BDT, xm) + b_c, 0.0)
        seg = seg_ref[...]
        oh = (_iota2((_B, _BN), 0)
              == jnp.broadcast_to(seg, (_B, _BN))).astype(_NT)
        f = _dotT(em, oh)
        cnt = _dotT((1.0 - m), oh)
        return f, cnt

    fn, cn = agg(xn_ref, cn_ref, segn_ref, BDnT_ref[...],
                 bcol_ref[:, 0:1], 2)
    fs, cs = agg(xs_ref, cs_ref, segs_ref, BDsT_ref[...],
                 bcol_ref[:, 1:2], 1)

    @pl.when(i == 0)
    def _():
        Fnb_ref[...] = fn
        Fsg_ref[...] = fs
        cntn_ref[...] = cn
        cnts_ref[...] = cs

    @pl.when(i > 0)
    def _():
        Fnb_ref[...] += fn
        Fsg_ref[...] += fs
        cntn_ref[...] += cn
        cnts_ref[...] += cs

    @pl.when(i == pl.num_programs(0) - 1)
    def _():
        S = _s512(32)
        ST = _s512T(32)
        km512 = ((_iota2((512, 512), 0) // 32)
                 == (_iota2((512, 512), 1) // 32)).astype(_F32)

        def bd(wT):
            return _dot(_dot(S, wT), ST) * km512

        w1T = w1T_ref[...]
        maskE = ((_iota2((512, _B), 0) // 32)
                 == _iota2((512, _B), 1)).astype(_F32)
        ErbnT = jnp.maximum(bcol_ref[:, 0:1], 0.0) * maskE
        ErbsT = jnp.maximum(bcol_ref[:, 1:2], 0.0) * maskE
        Fn = Fnb_ref[...] - _dot(ErbnT, cntn_ref[...])
        Fs = Fsg_ref[...] - _dot(ErbsT, cnts_ref[...])
        Fnb_ref[...] = Fn
        Fsg_ref[...] = Fs

        wtile_p = _dot(_dot(S, jnp.transpose(wp_ref[...])),
                       (_iota2((4, 64), 0)
                        == (_iota2((4, 64), 1) % 4)).astype(_F32))
        km64 = ((_iota2((512, 64), 0) // 32)
                == (_iota2((512, 64), 1) // 4)).astype(_F32)
        BDpT = wtile_p * km64
        P = jnp.maximum(_dot(BDpT, xped_ref[...]) + bcol_ref[:, 2:3], 0.0)
        P_ref[...] = P

        A1 = jnp.maximum(
            bd(w1T[:, 0:32]) @ P + bd(w1T[:, 32:64]) @ Fn
            + bd(w1T[:, 64:96]) @ Fs + bcol_ref[:, 3:4], 0.0)
        A2 = jnp.maximum(bd(jnp.transpose(w2_ref[...])) @ A1
                         + bcol_ref[:, 4:5], 0.0)
        w3L = _dot(w3T_ref[...], ST)
        BD3T = (jnp.broadcast_to(w3L, (_B, 512))
                * (_iota2((_B, 512), 0)
                   == (_iota2((_B, 512), 1) // 32)).astype(_F32))
        b3v = jnp.broadcast_to(bstack_ref[5:6, 0:1], (_T, _B))
        S_ref[...] = _dot(BD3T, A2) + b3v


@functools.partial(jax.jit, static_argnames=("interpret",))
def _run(x_ped, x_neighbor, x_sign, cls_neighbor, cls_sign,
         seg_neighbor, seg_sign,
         W_ped, b_ped, W_nb, b_nb, W_sg, b_sg,
         W1, b1, W2, b2, W3, b3, interpret=False):
    NB = _N // _BN

    NC = _N // 128
    xn4 = (x_neighbor.transpose(1, 0, 2).reshape(_T, NC, 128, 4)
           .transpose(0, 1, 3, 2))
    xs4 = (x_sign.transpose(1, 0, 2).reshape(_T, NC, 128, 2)
           .transpose(0, 1, 3, 2))
    cnT = cls_neighbor.T
    csT = cls_sign.T
    segn2 = seg_neighbor.reshape(1, _N)
    segs2 = seg_sign.reshape(1, _N)
    xpedT = x_ped.transpose(1, 2, 0).reshape(_T * 4, _B)
    bstack = jnp.stack([b_nb, b_sg, b_ped, b1, b2,
                        jnp.pad(b3, (0, 31))])

    const = lambda shape: pl.BlockSpec(shape, lambda i: (0,) * len(shape))
    Fnb, Fsg, P, S = pl.pallas_call(
        _body,
        grid=(NB,),
        in_specs=[
            pl.BlockSpec((_T, _CB, 4, 128), lambda i: (0, i, 0, 0)),
            pl.BlockSpec((_T, _BN), lambda i: (0, i)),
            pl.BlockSpec((1, _BN), lambda i: (0, i)),
            pl.BlockSpec((_T, _CB, 2, 128), lambda i: (0, i, 0, 0)),
            pl.BlockSpec((_T, _BN), lambda i: (0, i)),
            pl.BlockSpec((1, _BN), lambda i: (0, i)),
            const((32, 96)),
            const((32, 32)),
            const((1, 32)),
            const((4, 32)),
            const((4, 32)),
            const((2, 32)),
            const((6, 32)),
            const((_T * 4, _B)),
        ],
        out_specs=[
            const((512, _B)), const((512, _B)), const((512, _B)),
            const((_T, _B)),
        ],
        out_shape=[
            jax.ShapeDtypeStruct((512, _B), jnp.float32),
            jax.ShapeDtypeStruct((512, _B), jnp.float32),
            jax.ShapeDtypeStruct((512, _B), jnp.float32),
            jax.ShapeDtypeStruct((_T, _B), jnp.float32),
        ],
        scratch_shapes=[
            pltpu.VMEM((_T, _B), jnp.float32),
            pltpu.VMEM((_T, _B), jnp.float32),
            pltpu.VMEM((512, 64), jnp.float32),
            pltpu.VMEM((512, 32), jnp.float32),
            pltpu.VMEM((512, 8), jnp.float32),
        ],
        interpret=interpret,
    )(xn4, cnT, segn2, xs4, csT, segs2,
      W1.T, W2, W3.T, W_nb, W_ped, W_sg, bstack, xpedT)

    int_det_score = S.T.reshape(_B, _T, 1)
    all_traffic = jnp.concatenate(
        [P.T.reshape(_B, _T, 32), Fnb.T.reshape(_B, _T, 32),
         Fsg.T.reshape(_B, _T, 32)], axis=-1)
    return (int_det_score, all_traffic)


def kernel(x_ped, x_neighbor, x_sign, cls_neighbor, cls_sign,
           seg_neighbor, seg_sign,
           W_ped, b_ped, W_nb, b_nb, W_sg, b_sg,
           W1, b1, W2, b2, W3, b3):
    return _run(x_ped, x_neighbor, x_sign, cls_neighbor, cls_sign,
                seg_neighbor, seg_sign,
                W_ped, b_ped, W_nb, b_nb, W_sg, b_sg,
                W1, b1, W2, b2, W3, b3)
